# baseline (speedup 1.0000x reference)
import numpy as np

import jax
import jax.numpy as jnp
from jax.experimental import pallas as pl
from jax.experimental.pallas import tpu as pltpu
from jax.experimental.shard_map import shard_map
from jax.sharding import Mesh, PartitionSpec as P

N = 16
C = 4
H1 = 32
H2 = 16
HL = 8
BLK = 4096


def _body(x_ref, m1_ref, m2_ref, m3_ref, r_ref, aux_ref, o_ref):
    f32 = jnp.float32
    bf16 = jnp.bfloat16
    x = x_ref[...].astype(bf16)
    z1 = jnp.dot(x, m1_ref[...], preferred_element_type=f32)
    h1 = jnp.maximum(z1 + aux_ref[0:1, :], 0.0).astype(bf16)
    z2 = jnp.dot(h1, m2_ref[...], preferred_element_type=f32)
    h2 = jnp.maximum(z2 + aux_ref[1:2, 0:256], 0.0).astype(bf16)
    g = jnp.dot(h2, m3_ref[...], preferred_element_type=f32)
    g = g + aux_ref[2:3, 0:384]
    i_g = jax.nn.sigmoid(g[:, 0:128])
    g_g = jnp.tanh(g[:, 128:256])
    o_g = jax.nn.sigmoid(g[:, 256:384])
    h = (o_g * jnp.tanh(i_g * g_g)).astype(bf16)
    y = jnp.dot(h, r_ref[...], preferred_element_type=f32)
    o_ref[...] = y + aux_ref[3:4, 0:16]


def kernel(a_hat, x_b, w1, b1, w2, b2, wih, bih, bhh, wout, bout):
    f32 = jnp.float32
    bf16 = jnp.bfloat16
    B = x_b.shape[0]
    xv = x_b.reshape(B, N * C)
    pad = (-B) % BLK
    if pad:
        xv = jnp.concatenate([xv, jnp.zeros((pad, N * C), xv.dtype)], axis=0)
    nb = (B + pad) // BLK

    eye = jnp.eye(N, dtype=f32)
    m1 = jnp.einsum('nm,cj->mcnj', a_hat, w1).reshape(N * C, N * H1)
    m2 = jnp.einsum('nm,cj->mcnj', a_hat, w2).reshape(N * H1, N * H2)
    wg = jnp.concatenate([wih[:, 0:HL], wih[:, 2 * HL:4 * HL]], axis=1)
    m3 = jnp.einsum('ck,mn->mckn', wg, eye).reshape(N * H2, N * 3 * HL)
    r = jnp.einsum('j,nm->jnm', wout[:, 0], eye).reshape(N * HL, N)

    b1v = jnp.tile(b1[0], N)
    b2v = jnp.tile(b2[0], N)
    bg = (bih + bhh)[0]
    bgv = jnp.repeat(jnp.concatenate([bg[0:HL], bg[2 * HL:4 * HL]]), N)
    aux = jnp.zeros((8, N * H1), f32)
    aux = aux.at[0, :].set(b1v)
    aux = aux.at[1, 0:N * H2].set(b2v)
    aux = aux.at[2, 0:N * 3 * HL].set(bgv)
    aux = aux.at[3, :].set(bout[0, 0])

    def call(xv_l, m1_l, m2_l, m3_l, r_l, aux_l):
        nb_l = xv_l.shape[0] // BLK
        return pl.pallas_call(
            _body,
            out_shape=jax.ShapeDtypeStruct((xv_l.shape[0], N), f32),
            grid=(nb_l,),
            in_specs=[
                pl.BlockSpec((BLK, N * C), lambda i: (i, 0)),
                pl.BlockSpec((N * C, N * H1), lambda i: (0, 0)),
                pl.BlockSpec((N * H1, N * H2), lambda i: (0, 0)),
                pl.BlockSpec((N * H2, N * 3 * HL), lambda i: (0, 0)),
                pl.BlockSpec((N * HL, N), lambda i: (0, 0)),
                pl.BlockSpec((8, N * H1), lambda i: (0, 0)),
            ],
            out_specs=pl.BlockSpec((BLK, N), lambda i: (i, 0)),
            compiler_params=pltpu.CompilerParams(
                dimension_semantics=("parallel",)),
        )(xv_l, m1_l, m2_l, m3_l, r_l, aux_l)

    args = (xv, m1.astype(bf16), m2.astype(bf16), m3.astype(bf16),
            r.astype(bf16), aux)

    devs = jax.devices()
    nd = len(devs)
    while nd > 1 and (nb % nd or nd > nb):
        nd -= 1
    if nd > 1:
        mesh = Mesh(np.array(devs[:nd]), ("d",))
        reps = (P(None, None),) * 5
        out = shard_map(call, mesh=mesh,
                        in_specs=(P("d", None),) + reps,
                        out_specs=P("d", None),
                        check_rep=False)(*args)
    else:
        out = call(*args)
    return out[:B]


# speedup vs baseline: 2.5334x; 2.5334x over previous
import jax
import jax.numpy as jnp
from jax.experimental import pallas as pl
from jax.experimental.pallas import tpu as pltpu

N = 16
C = 4
H1 = 32
H2 = 16
HL = 8
BLK = 4096


def _body(x_ref, m1_ref, m2_ref, m3_ref, r_ref, aux_ref, o_ref):
    f32 = jnp.float32
    bf16 = jnp.bfloat16
    x = x_ref[...]
    z1 = jnp.dot(x, m1_ref[...], preferred_element_type=f32)
    h1 = jnp.maximum(z1 + aux_ref[0:1, :], 0.0).astype(bf16)
    z2 = jnp.dot(h1, m2_ref[...], preferred_element_type=f32)
    h2 = jnp.maximum(z2 + aux_ref[1:2, 0:256], 0.0).astype(bf16)
    g = jnp.dot(h2, m3_ref[...], preferred_element_type=f32)
    g = g + aux_ref[2:3, 0:384]
    i_g = jax.nn.sigmoid(g[:, 0:128])
    g_g = jnp.tanh(g[:, 128:256])
    o_g = jax.nn.sigmoid(g[:, 256:384])
    h = (o_g * jnp.tanh(i_g * g_g)).astype(bf16)
    y = jnp.dot(h, r_ref[...], preferred_element_type=f32)
    o_ref[...] = y + aux_ref[3:4, 0:16]


def kernel(a_hat, x_b, w1, b1, w2, b2, wih, bih, bhh, wout, bout):
    f32 = jnp.float32
    bf16 = jnp.bfloat16
    B = x_b.shape[0]
    xv = x_b.astype(bf16).reshape(B, N * C)
    pad = (-B) % BLK
    if pad:
        xv = jnp.concatenate([xv, jnp.zeros((pad, N * C), xv.dtype)], axis=0)
    nb = (B + pad) // BLK

    eye = jnp.eye(N, dtype=f32)
    m1 = jnp.einsum('nm,cj->mcnj', a_hat, w1).reshape(N * C, N * H1)
    m2 = jnp.einsum('nm,cj->mcnj', a_hat, w2).reshape(N * H1, N * H2)
    wg = jnp.concatenate([wih[:, 0:HL], wih[:, 2 * HL:4 * HL]], axis=1)
    m3 = jnp.einsum('ck,mn->mckn', wg, eye).reshape(N * H2, N * 3 * HL)
    r = jnp.einsum('j,nm->jnm', wout[:, 0], eye).reshape(N * HL, N)

    b1v = jnp.tile(b1[0], N)
    b2v = jnp.tile(b2[0], N)
    bg = (bih + bhh)[0]
    bgv = jnp.repeat(jnp.concatenate([bg[0:HL], bg[2 * HL:4 * HL]]), N)
    aux = jnp.zeros((8, N * H1), f32)
    aux = aux.at[0, :].set(b1v)
    aux = aux.at[1, 0:N * H2].set(b2v)
    aux = aux.at[2, 0:N * 3 * HL].set(bgv)
    aux = aux.at[3, :].set(bout[0, 0])

    def call(xv_l, m1_l, m2_l, m3_l, r_l, aux_l):
        nb_l = xv_l.shape[0] // BLK
        return pl.pallas_call(
            _body,
            out_shape=jax.ShapeDtypeStruct((xv_l.shape[0], N), f32),
            grid=(nb_l,),
            in_specs=[
                pl.BlockSpec((BLK, N * C), lambda i: (i, 0)),
                pl.BlockSpec((N * C, N * H1), lambda i: (0, 0)),
                pl.BlockSpec((N * H1, N * H2), lambda i: (0, 0)),
                pl.BlockSpec((N * H2, N * 3 * HL), lambda i: (0, 0)),
                pl.BlockSpec((N * HL, N), lambda i: (0, 0)),
                pl.BlockSpec((8, N * H1), lambda i: (0, 0)),
            ],
            out_specs=pl.BlockSpec((BLK, N), lambda i: (i, 0)),
            compiler_params=pltpu.CompilerParams(
                dimension_semantics=("parallel",)),
        )(xv_l, m1_l, m2_l, m3_l, r_l, aux_l)

    out = call(xv, m1.astype(bf16), m2.astype(bf16), m3.astype(bf16),
               r.astype(bf16), aux)
    return out[:B]


# speedup vs baseline: 2.5847x; 1.0202x over previous
import jax
import jax.numpy as jnp
from jax.experimental import pallas as pl
from jax.experimental.pallas import tpu as pltpu

N = 16
C = 4
H1 = 32
H2 = 16
HL = 8
BLK = 8192


def _sigmoid(x):
    return 0.5 * jnp.tanh(0.5 * x) + 0.5


def _body(x_ref, m1_ref, m2_ref, m3_ref, r_ref, aux_ref, auxh_ref, o_ref):
    f32 = jnp.float32
    bf16 = jnp.bfloat16
    x = x_ref[...]
    z1 = jnp.dot(x, m1_ref[...], preferred_element_type=f32)
    h1 = jnp.maximum(z1.astype(bf16) + auxh_ref[0:1, :], 0)
    z2 = jnp.dot(h1, m2_ref[...], preferred_element_type=f32)
    h2 = jnp.maximum(z2.astype(bf16) + auxh_ref[1:2, 0:256], 0)
    g = jnp.dot(h2, m3_ref[...], preferred_element_type=f32)
    g = g + aux_ref[2:3, 0:384]
    i_g = _sigmoid(g[:, 0:128])
    g_g = jnp.tanh(g[:, 128:256])
    o_g = _sigmoid(g[:, 256:384])
    h = (o_g * jnp.tanh(i_g * g_g)).astype(bf16)
    y = jnp.dot(h, r_ref[...], preferred_element_type=f32)
    o_ref[...] = y + aux_ref[3:4, 0:16]


def kernel(a_hat, x_b, w1, b1, w2, b2, wih, bih, bhh, wout, bout):
    f32 = jnp.float32
    bf16 = jnp.bfloat16
    B = x_b.shape[0]
    xv = x_b.astype(bf16).reshape(B, N * C)
    pad = (-B) % BLK
    if pad:
        xv = jnp.concatenate([xv, jnp.zeros((pad, N * C), xv.dtype)], axis=0)
    nb = (B + pad) // BLK

    eye = jnp.eye(N, dtype=f32)
    m1 = jnp.einsum('nm,cj->mcnj', a_hat, w1).reshape(N * C, N * H1)
    m2 = jnp.einsum('nm,cj->mcnj', a_hat, w2).reshape(N * H1, N * H2)
    wg = jnp.concatenate([wih[:, 0:HL], wih[:, 2 * HL:4 * HL]], axis=1)
    m3 = jnp.einsum('ck,mn->mckn', wg, eye).reshape(N * H2, N * 3 * HL)
    r = jnp.einsum('j,nm->jnm', wout[:, 0], eye).reshape(N * HL, N)

    b1v = jnp.tile(b1[0], N)
    b2v = jnp.tile(b2[0], N)
    bg = (bih + bhh)[0]
    bgv = jnp.repeat(jnp.concatenate([bg[0:HL], bg[2 * HL:4 * HL]]), N)
    aux = jnp.zeros((8, N * H1), f32)
    aux = aux.at[0, :].set(b1v)
    aux = aux.at[1, 0:N * H2].set(b2v)
    aux = aux.at[2, 0:N * 3 * HL].set(bgv)
    aux = aux.at[3, :].set(bout[0, 0])

    def call(xv_l, m1_l, m2_l, m3_l, r_l, aux_l, auxh_l):
        nb_l = xv_l.shape[0] // BLK
        return pl.pallas_call(
            _body,
            out_shape=jax.ShapeDtypeStruct((xv_l.shape[0], N), f32),
            grid=(nb_l,),
            in_specs=[
                pl.BlockSpec((BLK, N * C), lambda i: (i, 0)),
                pl.BlockSpec((N * C, N * H1), lambda i: (0, 0)),
                pl.BlockSpec((N * H1, N * H2), lambda i: (0, 0)),
                pl.BlockSpec((N * H2, N * 3 * HL), lambda i: (0, 0)),
                pl.BlockSpec((N * HL, N), lambda i: (0, 0)),
                pl.BlockSpec((8, N * H1), lambda i: (0, 0)),
                pl.BlockSpec((8, N * H1), lambda i: (0, 0)),
            ],
            out_specs=pl.BlockSpec((BLK, N), lambda i: (i, 0)),
            compiler_params=pltpu.CompilerParams(
                dimension_semantics=("parallel",)),
        )(xv_l, m1_l, m2_l, m3_l, r_l, aux_l, auxh_l)

    out = call(xv, m1.astype(bf16), m2.astype(bf16), m3.astype(bf16),
               r.astype(bf16), aux, aux.astype(bf16))
    return out[:B]


# speedup vs baseline: 2.5852x; 1.0002x over previous
import jax
import jax.numpy as jnp
from jax.experimental import pallas as pl
from jax.experimental.pallas import tpu as pltpu

N = 16
C = 4
H1 = 32
H2 = 16
HL = 8
BLK = 16384


def _sigmoid(x):
    return 0.5 * jnp.tanh(0.5 * x) + 0.5


def _body(x_ref, m1_ref, m2_ref, m3_ref, r_ref, aux_ref, auxh_ref, o_ref):
    f32 = jnp.float32
    bf16 = jnp.bfloat16
    x = x_ref[...]
    z1 = jnp.dot(x, m1_ref[...], preferred_element_type=f32)
    h1 = jnp.maximum(z1.astype(bf16) + auxh_ref[0:1, :], 0)
    z2 = jnp.dot(h1, m2_ref[...], preferred_element_type=f32)
    h2 = jnp.maximum(z2.astype(bf16) + auxh_ref[1:2, 0:256], 0)
    g = jnp.dot(h2, m3_ref[...], preferred_element_type=f32)
    g = g + aux_ref[2:3, 0:384]
    i_g = _sigmoid(g[:, 0:128])
    g_g = jnp.tanh(g[:, 128:256])
    o_g = _sigmoid(g[:, 256:384])
    h = (o_g * jnp.tanh(i_g * g_g)).astype(bf16)
    y = jnp.dot(h, r_ref[...], preferred_element_type=f32)
    o_ref[...] = y + aux_ref[3:4, 0:16]


def kernel(a_hat, x_b, w1, b1, w2, b2, wih, bih, bhh, wout, bout):
    f32 = jnp.float32
    bf16 = jnp.bfloat16
    B = x_b.shape[0]
    xv = x_b.astype(bf16).reshape(B, N * C)
    pad = (-B) % BLK
    if pad:
        xv = jnp.concatenate([xv, jnp.zeros((pad, N * C), xv.dtype)], axis=0)
    nb = (B + pad) // BLK

    eye = jnp.eye(N, dtype=f32)
    m1 = jnp.einsum('nm,cj->mcnj', a_hat, w1).reshape(N * C, N * H1)
    m2 = jnp.einsum('nm,cj->mcnj', a_hat, w2).reshape(N * H1, N * H2)
    wg = jnp.concatenate([wih[:, 0:HL], wih[:, 2 * HL:4 * HL]], axis=1)
    m3 = jnp.einsum('ck,mn->mckn', wg, eye).reshape(N * H2, N * 3 * HL)
    r = jnp.einsum('j,nm->jnm', wout[:, 0], eye).reshape(N * HL, N)

    b1v = jnp.tile(b1[0], N)
    b2v = jnp.tile(b2[0], N)
    bg = (bih + bhh)[0]
    bgv = jnp.repeat(jnp.concatenate([bg[0:HL], bg[2 * HL:4 * HL]]), N)
    aux = jnp.zeros((8, N * H1), f32)
    aux = aux.at[0, :].set(b1v)
    aux = aux.at[1, 0:N * H2].set(b2v)
    aux = aux.at[2, 0:N * 3 * HL].set(bgv)
    aux = aux.at[3, :].set(bout[0, 0])
    aux = aux.at[4, 0:N * HL].set(jnp.repeat(wout[:, 0], N))

    def call(xv_l, m1_l, m2_l, m3_l, r_l, aux_l, auxh_l):
        nb_l = xv_l.shape[0] // BLK
        return pl.pallas_call(
            _body,
            out_shape=jax.ShapeDtypeStruct((xv_l.shape[0], N), f32),
            grid=(nb_l,),
            in_specs=[
                pl.BlockSpec((BLK, N * C), lambda i: (i, 0)),
                pl.BlockSpec((N * C, N * H1), lambda i: (0, 0)),
                pl.BlockSpec((N * H1, N * H2), lambda i: (0, 0)),
                pl.BlockSpec((N * H2, N * 3 * HL), lambda i: (0, 0)),
                pl.BlockSpec((N * HL, N), lambda i: (0, 0)),
                pl.BlockSpec((8, N * H1), lambda i: (0, 0)),
                pl.BlockSpec((8, N * H1), lambda i: (0, 0)),
            ],
            out_specs=pl.BlockSpec((BLK, N), lambda i: (i, 0)),
            compiler_params=pltpu.CompilerParams(
                dimension_semantics=("parallel",)),
        )(xv_l, m1_l, m2_l, m3_l, r_l, aux_l, auxh_l)

    out = call(xv, m1.astype(bf16), m2.astype(bf16), m3.astype(bf16),
               r.astype(bf16), aux, aux.astype(bf16))
    return out[:B]


# speedup vs baseline: 3.8604x; 1.4933x over previous
import jax
import jax.numpy as jnp
from jax.experimental import pallas as pl
from jax.experimental.pallas import tpu as pltpu

N = 16
C = 4
H1 = 32
H2 = 16
HL = 8
BLKL = 8192

_B1 = 0
_B2 = _B1 + N * H1
_BG = _B2 + N * H2
_WO = _BG + N * 3 * HL
_BO = _WO + N * HL
AUXR = _BO + N + 8 - (_BO + N) % 8


def _sigmoid(x):
    return 0.5 * jnp.tanh(0.5 * x) + 0.5


def _body(x_ref, m1_ref, m2_ref, m3_ref, aux_ref, auxh_ref, o_ref):
    f32 = jnp.float32
    bf16 = jnp.bfloat16
    x = x_ref[...]
    z1 = jnp.dot(m1_ref[...], x, preferred_element_type=f32)
    h1 = jnp.maximum(z1.astype(bf16) + auxh_ref[_B1:_B2, :], 0)
    z2 = jnp.dot(m2_ref[...], h1, preferred_element_type=f32)
    h2 = jnp.maximum(z2.astype(bf16) + auxh_ref[_B2:_BG, :], 0)
    g = jnp.dot(m3_ref[...], h2, preferred_element_type=f32)
    g = g + aux_ref[_BG:_WO, :]
    i_g = _sigmoid(g[0:128, :])
    g_g = jnp.tanh(g[128:256, :])
    o_g = _sigmoid(g[256:384, :])
    h = o_g * jnp.tanh(i_g * g_g)
    y = aux_ref[_BO:_BO + N, :]
    for j in range(HL):
        y = y + aux_ref[_WO + j * N:_WO + (j + 1) * N, :] * h[j * N:(j + 1) * N, :]
    o_ref[...] = y


def kernel(a_hat, x_b, w1, b1, w2, b2, wih, bih, bhh, wout, bout):
    f32 = jnp.float32
    bf16 = jnp.bfloat16
    B = x_b.shape[0]
    xt = x_b.astype(bf16).transpose(1, 2, 0).reshape(N * C, B)
    pad = (-B) % BLKL
    if pad:
        xt = jnp.concatenate([xt, jnp.zeros((N * C, pad), xt.dtype)], axis=1)
    nb = (B + pad) // BLKL

    eye = jnp.eye(N, dtype=f32)
    m1 = jnp.einsum('nm,cj->njmc', a_hat, w1).reshape(N * H1, N * C)
    m2 = jnp.einsum('nm,cj->njmc', a_hat, w2).reshape(N * H2, N * H1)
    wg = jnp.concatenate([wih[:, 0:HL], wih[:, 2 * HL:4 * HL]], axis=1)
    m3 = jnp.einsum('ck,mn->knmc', wg, eye).reshape(N * 3 * HL, N * H2)

    bg = (bih + bhh)[0]
    bg_igo = jnp.concatenate([bg[0:HL], bg[2 * HL:4 * HL]])
    aux = jnp.zeros((AUXR, 1), f32)
    aux = aux.at[_B1:_B2, 0].set(jnp.tile(b1[0], N))
    aux = aux.at[_B2:_BG, 0].set(jnp.tile(b2[0], N))
    aux = aux.at[_BG:_WO, 0].set(jnp.repeat(bg_igo, N))
    aux = aux.at[_WO:_BO, 0].set(jnp.repeat(wout[:, 0], N))
    aux = aux.at[_BO:_BO + N, 0].set(bout[0, 0])

    out = pl.pallas_call(
        _body,
        out_shape=jax.ShapeDtypeStruct((N, B + pad), f32),
        grid=(nb,),
        in_specs=[
            pl.BlockSpec((N * C, BLKL), lambda i: (0, i)),
            pl.BlockSpec((N * H1, N * C), lambda i: (0, 0)),
            pl.BlockSpec((N * H2, N * H1), lambda i: (0, 0)),
            pl.BlockSpec((N * 3 * HL, N * H2), lambda i: (0, 0)),
            pl.BlockSpec((AUXR, 1), lambda i: (0, 0)),
            pl.BlockSpec((AUXR, 1), lambda i: (0, 0)),
        ],
        out_specs=pl.BlockSpec((N, BLKL), lambda i: (0, i)),
        compiler_params=pltpu.CompilerParams(
            dimension_semantics=("parallel",)),
    )(xt, m1.astype(bf16), m2.astype(bf16), m3.astype(bf16),
      aux, aux.astype(bf16))
    return out[:, :B].T


# speedup vs baseline: 3.8730x; 1.0033x over previous
import jax
import jax.numpy as jnp
from jax.experimental import pallas as pl
from jax.experimental.pallas import tpu as pltpu

N = 16
C = 4
H1 = 32
H2 = 16
HL = 8
BLKL = 16384

_B1 = 0
_B2 = _B1 + N * H1
_BG = _B2 + N * H2
_WO = _BG + N * 3 * HL
_BO = _WO + N * HL
AUXR = _BO + N + 8 - (_BO + N) % 8


def _sigmoid(x):
    return 0.5 * jnp.tanh(0.5 * x) + 0.5


def _body(x_ref, m1_ref, m2_ref, m3_ref, aux_ref, auxh_ref, o_ref):
    f32 = jnp.float32
    bf16 = jnp.bfloat16
    x = x_ref[...]
    z1 = jnp.dot(m1_ref[...], x, preferred_element_type=f32)
    h1 = jnp.maximum(z1.astype(bf16) + auxh_ref[_B1:_B2, :], 0)
    z2 = jnp.dot(m2_ref[...], h1, preferred_element_type=f32)
    h2 = jnp.maximum(z2.astype(bf16) + auxh_ref[_B2:_BG, :], 0)
    g = jnp.dot(m3_ref[...], h2, preferred_element_type=f32)
    g = g + aux_ref[_BG:_WO, :]
    i_g = _sigmoid(g[0:128, :])
    g_g = jnp.tanh(g[128:256, :])
    o_g = _sigmoid(g[256:384, :])
    h = o_g * jnp.tanh(i_g * g_g)
    y = aux_ref[_BO:_BO + N, :]
    for j in range(HL):
        y = y + aux_ref[_WO + j * N:_WO + (j + 1) * N, :] * h[j * N:(j + 1) * N, :]
    o_ref[...] = y


def kernel(a_hat, x_b, w1, b1, w2, b2, wih, bih, bhh, wout, bout):
    f32 = jnp.float32
    bf16 = jnp.bfloat16
    B = x_b.shape[0]
    xt = x_b.astype(bf16).transpose(1, 2, 0).reshape(N * C, B)
    pad = (-B) % BLKL
    if pad:
        xt = jnp.concatenate([xt, jnp.zeros((N * C, pad), xt.dtype)], axis=1)
    nb = (B + pad) // BLKL

    eye = jnp.eye(N, dtype=f32)
    m1 = jnp.einsum('nm,cj->njmc', a_hat, w1).reshape(N * H1, N * C)
    m2 = jnp.einsum('nm,cj->njmc', a_hat, w2).reshape(N * H2, N * H1)
    wg = jnp.concatenate([wih[:, 0:HL], wih[:, 2 * HL:4 * HL]], axis=1)
    m3 = jnp.einsum('ck,mn->knmc', wg, eye).reshape(N * 3 * HL, N * H2)

    bg = (bih + bhh)[0]
    bg_igo = jnp.concatenate([bg[0:HL], bg[2 * HL:4 * HL]])
    aux = jnp.zeros((AUXR, 1), f32)
    aux = aux.at[_B1:_B2, 0].set(jnp.tile(b1[0], N))
    aux = aux.at[_B2:_BG, 0].set(jnp.tile(b2[0], N))
    aux = aux.at[_BG:_WO, 0].set(jnp.repeat(bg_igo, N))
    aux = aux.at[_WO:_BO, 0].set(jnp.repeat(wout[:, 0], N))
    aux = aux.at[_BO:_BO + N, 0].set(bout[0, 0])

    out = pl.pallas_call(
        _body,
        out_shape=jax.ShapeDtypeStruct((N, B + pad), f32),
        grid=(nb,),
        in_specs=[
            pl.BlockSpec((N * C, BLKL), lambda i: (0, i)),
            pl.BlockSpec((N * H1, N * C), lambda i: (0, 0)),
            pl.BlockSpec((N * H2, N * H1), lambda i: (0, 0)),
            pl.BlockSpec((N * 3 * HL, N * H2), lambda i: (0, 0)),
            pl.BlockSpec((AUXR, 1), lambda i: (0, 0)),
            pl.BlockSpec((AUXR, 1), lambda i: (0, 0)),
        ],
        out_specs=pl.BlockSpec((N, BLKL), lambda i: (0, i)),
        compiler_params=pltpu.CompilerParams(
            dimension_semantics=("parallel",)),
    )(xt, m1.astype(bf16), m2.astype(bf16), m3.astype(bf16),
      aux, aux.astype(bf16))
    return out[:, :B].T
